# baseline (speedup 1.0000x reference)
import functools
import math

import jax
import jax.numpy as jnp
import numpy as np
from jax.experimental import pallas as pl
from jax.experimental.pallas import tpu as pltpu

DIM = 1024
HEADS = 8
DIM_HEAD = 64
INNER = HEADS * DIM_HEAD
K_Q = 10
L = 32768
K_TOP = int(np.ceil(L * 0.8))
N_KEEP = L - K_TOP
BLK = 256
NBLK = L // BLK
ROWS1 = 256
LANES1 = 128
NEG = -2.0e30
MFLOOR = -1.0e30
EPS = 1e-5


def _mask_kernel(attn_ref, gq_ref, nw_ref, nb_ref, tqw_ref, wk_ref,
                 keep_ref, bias_ref, dst_ref, qwn_ref):
    a = attn_ref[...]
    ui = jax.lax.bitcast_convert_type(a, jnp.uint32)
    sign = ui >= jnp.uint32(0x80000000)
    u = jnp.where(sign, ~ui, ui | jnp.uint32(0x80000000))

    kth = jnp.int32(K_TOP - 1)

    def body(b, p):
        t = p | (jax.lax.shift_right_logical(
            jnp.uint32(0x80000000), jnp.uint32(b)))
        cnt = jnp.sum((u < t).astype(jnp.int32))
        return jax.lax.select(cnt <= kth, t, p)

    thr = jax.lax.fori_loop(0, 32, body, jnp.uint32(0))

    less = u < thr
    eq = u == thr
    c1 = jnp.sum(less.astype(jnp.int32))
    need = jnp.int32(K_TOP) - c1

    eqf = eq.astype(jnp.float32)
    iu = jax.lax.broadcasted_iota(jnp.int32, (LANES1, LANES1), 0)
    ju = jax.lax.broadcasted_iota(jnp.int32, (LANES1, LANES1), 1)
    U128 = (iu < ju).astype(jnp.float32)
    rowcum = jnp.dot(eqf, U128, preferred_element_type=jnp.float32)
    il = jax.lax.broadcasted_iota(jnp.int32, (ROWS1, ROWS1), 0)
    jl = jax.lax.broadcasted_iota(jnp.int32, (ROWS1, ROWS1), 1)
    L256 = (jl < il).astype(jnp.float32)
    ones_l = jnp.ones((LANES1, 1), jnp.float32)
    et = jnp.dot(eqf, ones_l, preferred_element_type=jnp.float32)
    eoff = jnp.dot(L256, et, preferred_element_type=jnp.float32)
    eq_prefix = rowcum + eoff

    masked = less | (eq & (eq_prefix < need.astype(jnp.float32)))
    keepf = jnp.where(masked, 0.0, 1.0).astype(jnp.float32)
    keep_ref[...] = keepf
    bias_ref[...] = jnp.where(masked, 0.0, NEG).astype(jnp.float32)

    kt = jnp.dot(keepf, ones_l, preferred_element_type=jnp.float32)
    ko = jnp.dot(L256, kt, preferred_element_type=jnp.float32)
    rpb = BLK // LANES1
    dst_ref[...] = ko.reshape(NBLK, rpb)[:, 0:1].astype(jnp.int32)

    gq = gq_ref[...]
    nw = nw_ref[...]
    nb = nb_ref[...]
    mu = jnp.mean(gq, axis=1, keepdims=True)
    var = jnp.mean((gq - mu) ** 2, axis=1, keepdims=True)
    qn = (gq - mu) * jax.lax.rsqrt(var + EPS) * nw + nb
    q_ = jax.lax.dot_general(qn, tqw_ref[...], (((1,), (1,)), ((), ())),
                             preferred_element_type=jnp.float32)
    scale = DIM_HEAD ** (-0.5)
    parts = []
    for h in range(HEADS):
        qh = q_[:, h * DIM_HEAD:(h + 1) * DIM_HEAD]
        wkh = wk_ref[h * DIM_HEAD:(h + 1) * DIM_HEAD, :]
        parts.append(jax.lax.dot_general(
            qh, wkh, (((1,), (0,)), ((), ())),
            preferred_element_type=jnp.float32))
    qwn_ref[...] = jnp.concatenate(parts, axis=0) * scale


def _main_kernel(dst_ref, x_ref, keep_ref, bias_ref, qwn_ref, wv_ref,
                 wout_ref, outb_ref, nw_ref, nb_ref,
                 out_ref,
                 m_ref, s_ref, acc_ref, xk_ref, mg_ref, bnd_ref, sem):
    j = pl.program_id(0)

    @pl.when(j == 0)
    def _init():
        m_ref[...] = jnp.full_like(m_ref, MFLOOR)
        s_ref[...] = jnp.zeros_like(s_ref)
        acc_ref[...] = jnp.zeros_like(acc_ref)

    xb = x_ref[...]
    keep = keep_ref[...].reshape(1, BLK)
    bias = bias_ref[...].reshape(1, BLK)

    dst = dst_ref[j]
    r = jax.lax.rem(dst, 8)
    dst8 = pl.multiple_of(dst - r, 8)
    cpin = pltpu.make_async_copy(out_ref.at[pl.ds(dst8, 8), :], bnd_ref, sem)
    cpin.start()

    mu = jnp.mean(xb, axis=1, keepdims=True)
    var = jnp.mean((xb - mu) ** 2, axis=1, keepdims=True)
    lnx = (xb - mu) * jax.lax.rsqrt(var + EPS) * nw_ref[...] + nb_ref[...]

    dots = jax.lax.dot_general(qwn_ref[...], lnx, (((1,), (1,)), ((), ())),
                               preferred_element_type=jnp.float32)
    dots = dots + bias

    m_old = m_ref[:, 0:1]
    m_new = jnp.maximum(m_old, jnp.max(dots, axis=1, keepdims=True))
    alpha = jnp.exp(m_old - m_new)
    p = jnp.exp(dots - m_new)
    s_ref[:, 0:1] = s_ref[:, 0:1] * alpha + jnp.sum(p, axis=1, keepdims=True)
    acc_ref[...] = acc_ref[...] * alpha + jax.lax.dot_general(
        p, lnx, (((1,), (0,)), ((), ())), preferred_element_type=jnp.float32)
    m_ref[:, 0:1] = m_new

    ibu = jax.lax.broadcasted_iota(jnp.int32, (BLK, BLK), 0)
    jbu = jax.lax.broadcasted_iota(jnp.int32, (BLK, BLK), 1)
    Ux = (ibu < jbu).astype(jnp.float32)
    dstl = jnp.dot(keep, Ux, preferred_element_type=jnp.float32)
    ib = jax.lax.broadcasted_iota(jnp.int32, (BLK + 8, BLK), 0)
    sel = ((ib.astype(jnp.float32) == dstl + r.astype(jnp.float32))
           * keep).astype(jnp.bfloat16)
    xh = xb.astype(jnp.bfloat16)
    xl = (xb - xh.astype(jnp.float32)).astype(jnp.bfloat16)
    xk = (jnp.dot(sel, xh, preferred_element_type=jnp.float32)
          + jnp.dot(sel, xl, preferred_element_type=jnp.float32))
    cpin.wait()
    rowi = jax.lax.broadcasted_iota(jnp.int32, (8, 1), 0)
    head = jnp.where(rowi < r, bnd_ref[...], xk[0:8, :])
    xk_ref[0:8, :] = head
    xk_ref[8:, :] = xk[8:, :]
    cp = pltpu.make_async_copy(
        xk_ref, out_ref.at[pl.ds(dst8, BLK + 8), :], sem)
    cp.start()
    cp.wait()

    @pl.when(j == NBLK - 1)
    def _epilogue():
        t = jax.lax.dot_general(acc_ref[...], wv_ref[...],
                                (((1,), (1,)), ((), ())),
                                preferred_element_type=jnp.float32)
        t = t / s_ref[:, 0:1]
        col = jax.lax.broadcasted_iota(jnp.int32, (1, INNER), 1) // DIM_HEAD
        o10 = jnp.zeros((K_Q, INNER), jnp.float32)
        for h in range(HEADS):
            o10 = o10 + t[h * K_Q:(h + 1) * K_Q, :] * (col == h).astype(
                jnp.float32)
        xm = jax.lax.dot_general(o10, wout_ref[...], (((1,), (1,)), ((), ())),
                                 preferred_element_type=jnp.float32)
        xm = xm + outb_ref[...]
        mb = (N_KEEP // 8) * 8
        mr = N_KEEP - mb
        cpin2 = pltpu.make_async_copy(
            out_ref.at[pl.ds(mb, 8), :], bnd_ref, sem)
        cpin2.start()
        cpin2.wait()
        mg_ref[...] = jnp.zeros_like(mg_ref)
        rowm = jax.lax.broadcasted_iota(jnp.int32, (8, 1), 0)
        mg_ref[0:8, :] = jnp.where(rowm < mr, bnd_ref[...],
                                   jnp.zeros((8, DIM), jnp.float32))
        mg_ref[mr:mr + K_Q, :] = xm
        cp2 = pltpu.make_async_copy(
            mg_ref, out_ref.at[pl.ds(mb, 24), :], sem)
        cp2.start()
        cp2.wait()


@jax.jit
def kernel(x, attn, global_q, norm_w, norm_b, to_kv_w, to_q_w, to_out_w,
           to_out_b):
    xs = x[0]
    a2 = attn.reshape(ROWS1, LANES1)
    gq = global_q[0]
    nw = norm_w.reshape(1, DIM)
    nb = norm_b.reshape(1, DIM)
    wk = to_kv_w[:INNER, :]
    wv = to_kv_w[INNER:, :]
    ob = to_out_b.reshape(1, DIM)

    keepf, biasf, dsts, qwn = pl.pallas_call(
        _mask_kernel,
        out_shape=(
            jax.ShapeDtypeStruct((ROWS1, LANES1), jnp.float32),
            jax.ShapeDtypeStruct((ROWS1, LANES1), jnp.float32),
            jax.ShapeDtypeStruct((NBLK, 1), jnp.int32),
            jax.ShapeDtypeStruct((HEADS * K_Q, DIM), jnp.float32),
        ),
    )(a2, gq, nw, nb, to_q_w, wk)

    keep3 = keepf.reshape(NBLK, 1, BLK)
    bias3 = biasf.reshape(NBLK, 1, BLK)
    dflat = dsts.reshape(NBLK)

    OUT_PAD = N_KEEP + BLK + 16
    grid_spec = pltpu.PrefetchScalarGridSpec(
        num_scalar_prefetch=1,
        grid=(NBLK,),
        in_specs=[
            pl.BlockSpec((BLK, DIM), lambda j, d: (j, 0)),
            pl.BlockSpec((1, 1, BLK), lambda j, d: (j, 0, 0)),
            pl.BlockSpec((1, 1, BLK), lambda j, d: (j, 0, 0)),
            pl.BlockSpec((HEADS * K_Q, DIM), lambda j, d: (0, 0)),
            pl.BlockSpec((INNER, DIM), lambda j, d: (0, 0)),
            pl.BlockSpec((DIM, INNER), lambda j, d: (0, 0)),
            pl.BlockSpec((1, DIM), lambda j, d: (0, 0)),
            pl.BlockSpec((1, DIM), lambda j, d: (0, 0)),
            pl.BlockSpec((1, DIM), lambda j, d: (0, 0)),
        ],
        out_specs=pl.BlockSpec(memory_space=pl.ANY),
        scratch_shapes=[
            pltpu.VMEM((HEADS * K_Q, 128), jnp.float32),
            pltpu.VMEM((HEADS * K_Q, 128), jnp.float32),
            pltpu.VMEM((HEADS * K_Q, DIM), jnp.float32),
            pltpu.VMEM((BLK + 8, DIM), jnp.float32),
            pltpu.VMEM((24, DIM), jnp.float32),
            pltpu.VMEM((8, DIM), jnp.float32),
            pltpu.SemaphoreType.DMA,
        ],
    )
    out_buf = pl.pallas_call(
        _main_kernel,
        grid_spec=grid_spec,
        out_shape=jax.ShapeDtypeStruct((OUT_PAD, DIM), jnp.float32),
        compiler_params=pltpu.CompilerParams(
            dimension_semantics=("arbitrary",)),
    )(dflat, xs, keep3, bias3, qwn, wv, to_out_w, ob, nw, nb)

    return out_buf[:N_KEEP + K_Q][None, :, :]


# speedup vs baseline: 1.1323x; 1.1323x over previous
import functools
import math

import jax
import jax.numpy as jnp
import numpy as np
from jax.experimental import pallas as pl
from jax.experimental.pallas import tpu as pltpu

DIM = 1024
HEADS = 8
DIM_HEAD = 64
INNER = HEADS * DIM_HEAD
K_Q = 10
L = 32768
K_TOP = int(np.ceil(L * 0.8))
N_KEEP = L - K_TOP
BLK = 256
NBLK = L // BLK
ROWS1 = 256
LANES1 = 128
NEG = -2.0e30
MFLOOR = -1.0e30
EPS = 1e-5


def _mask_kernel(attn_ref, gq_ref, nw_ref, nb_ref, tqw_ref, wk_ref,
                 keep_ref, bias_ref, dst_ref, qwn_ref):
    a = attn_ref[...]
    ui = jax.lax.bitcast_convert_type(a, jnp.uint32)
    sign = ui >= jnp.uint32(0x80000000)
    u = jnp.where(sign, ~ui, ui | jnp.uint32(0x80000000))

    kth = jnp.int32(K_TOP - 1)

    def body(b, p):
        t = p | (jax.lax.shift_right_logical(
            jnp.uint32(0x80000000), jnp.uint32(b)))
        cnt = jnp.sum((u < t).astype(jnp.int32))
        return jax.lax.select(cnt <= kth, t, p)

    thr = jax.lax.fori_loop(0, 32, body, jnp.uint32(0))

    less = u < thr
    eq = u == thr
    c1 = jnp.sum(less.astype(jnp.int32))
    need = jnp.int32(K_TOP) - c1

    eqf = eq.astype(jnp.float32)
    iu = jax.lax.broadcasted_iota(jnp.int32, (LANES1, LANES1), 0)
    ju = jax.lax.broadcasted_iota(jnp.int32, (LANES1, LANES1), 1)
    U128 = (iu < ju).astype(jnp.float32)
    rowcum = jnp.dot(eqf, U128, preferred_element_type=jnp.float32)
    il = jax.lax.broadcasted_iota(jnp.int32, (ROWS1, ROWS1), 0)
    jl = jax.lax.broadcasted_iota(jnp.int32, (ROWS1, ROWS1), 1)
    L256 = (jl < il).astype(jnp.float32)
    ones_l = jnp.ones((LANES1, 1), jnp.float32)
    et = jnp.dot(eqf, ones_l, preferred_element_type=jnp.float32)
    eoff = jnp.dot(L256, et, preferred_element_type=jnp.float32)
    eq_prefix = rowcum + eoff

    masked = less | (eq & (eq_prefix < need.astype(jnp.float32)))
    keepf = jnp.where(masked, 0.0, 1.0).astype(jnp.float32)
    keep_ref[...] = keepf
    bias_ref[...] = jnp.where(masked, 0.0, NEG).astype(jnp.float32)

    kt = jnp.dot(keepf, ones_l, preferred_element_type=jnp.float32)
    ko = jnp.dot(L256, kt, preferred_element_type=jnp.float32)
    rpb = BLK // LANES1
    dst_ref[...] = ko.reshape(NBLK, rpb)[:, 0:1].astype(jnp.int32)

    gq = gq_ref[...]
    nw = nw_ref[...]
    nb = nb_ref[...]
    mu = jnp.mean(gq, axis=1, keepdims=True)
    var = jnp.mean((gq - mu) ** 2, axis=1, keepdims=True)
    qn = (gq - mu) * jax.lax.rsqrt(var + EPS) * nw + nb
    q_ = jax.lax.dot_general(qn, tqw_ref[...], (((1,), (1,)), ((), ())),
                             preferred_element_type=jnp.float32)
    scale = DIM_HEAD ** (-0.5)
    parts = []
    for h in range(HEADS):
        qh = q_[:, h * DIM_HEAD:(h + 1) * DIM_HEAD]
        wkh = wk_ref[h * DIM_HEAD:(h + 1) * DIM_HEAD, :]
        parts.append(jax.lax.dot_general(
            qh, wkh, (((1,), (0,)), ((), ())),
            preferred_element_type=jnp.float32))
    qwn_ref[...] = jnp.concatenate(parts, axis=0) * scale


def _main_kernel(dst_ref, x_ref, keep_ref, bias_ref, qwn_ref, wv_ref,
                 wout_ref, outb_ref, nw_ref, nb_ref,
                 out_ref,
                 m_ref, s_ref, acc_ref, xk_ref, mg_ref, bnd_ref, sem):
    j = pl.program_id(0)

    @pl.when(j == 0)
    def _init():
        m_ref[...] = jnp.full_like(m_ref, MFLOOR)
        s_ref[...] = jnp.zeros_like(s_ref)
        acc_ref[...] = jnp.zeros_like(acc_ref)

    xb = x_ref[...]
    keep = keep_ref[...].reshape(1, BLK)
    bias = bias_ref[...].reshape(1, BLK)

    dst = dst_ref[j]
    r = jax.lax.rem(dst, 8)
    dst8 = pl.multiple_of(dst - r, 8)
    cpin = pltpu.make_async_copy(out_ref.at[pl.ds(dst8, 8), :], bnd_ref, sem)
    cpin.start()

    mu = jnp.mean(xb, axis=1, keepdims=True)
    var = jnp.mean((xb - mu) ** 2, axis=1, keepdims=True)
    inv = jax.lax.rsqrt(var + EPS)
    mu_t = jnp.transpose(mu)
    inv_t = jnp.transpose(inv)
    qwn = qwn_ref[...]
    nw = nw_ref[...]
    nb = nb_ref[...]
    qww = qwn * nw
    t1 = jnp.sum(qww, axis=1, keepdims=True)
    t2 = jnp.sum(qwn * nb, axis=1, keepdims=True)
    A = jax.lax.dot_general(qww, xb, (((1,), (1,)), ((), ())),
                            preferred_element_type=jnp.float32)
    dots = inv_t * (A - t1 * mu_t) + t2 + bias

    m_old = m_ref[:, 0:1]
    m_new = jnp.maximum(m_old, jnp.max(dots, axis=1, keepdims=True))
    alpha = jnp.exp(m_old - m_new)
    p = jnp.exp(dots - m_new)
    sp = jnp.sum(p, axis=1, keepdims=True)
    s_ref[:, 0:1] = s_ref[:, 0:1] * alpha + sp
    g = p * inv_t
    G = jax.lax.dot_general(g, xb, (((1,), (0,)), ((), ())),
                            preferred_element_type=jnp.float32)
    c1 = jnp.sum(g * mu_t, axis=1, keepdims=True)
    acc_ref[...] = acc_ref[...] * alpha + ((G - c1) * nw + sp * nb)
    m_ref[:, 0:1] = m_new

    ibu = jax.lax.broadcasted_iota(jnp.int32, (BLK, BLK), 0)
    jbu = jax.lax.broadcasted_iota(jnp.int32, (BLK, BLK), 1)
    Ux = (ibu < jbu).astype(jnp.float32)
    dstl = jnp.dot(keep, Ux, preferred_element_type=jnp.float32)
    ib = jax.lax.broadcasted_iota(jnp.int32, (BLK + 8, BLK), 0)
    sel = ((ib.astype(jnp.float32) == dstl + r.astype(jnp.float32))
           * keep).astype(jnp.float32)
    xk = jnp.dot(sel, xb, preferred_element_type=jnp.float32)
    cpin.wait()
    rowi = jax.lax.broadcasted_iota(jnp.int32, (8, 1), 0)
    head = jnp.where(rowi < r, bnd_ref[...], xk[0:8, :])
    xk_ref[0:8, :] = head
    xk_ref[8:, :] = xk[8:, :]
    cp = pltpu.make_async_copy(
        xk_ref, out_ref.at[pl.ds(dst8, BLK + 8), :], sem)
    cp.start()
    cp.wait()

    @pl.when(j == NBLK - 1)
    def _epilogue():
        t = jax.lax.dot_general(acc_ref[...], wv_ref[...],
                                (((1,), (1,)), ((), ())),
                                preferred_element_type=jnp.float32)
        t = t / s_ref[:, 0:1]
        col = jax.lax.broadcasted_iota(jnp.int32, (1, INNER), 1) // DIM_HEAD
        o10 = jnp.zeros((K_Q, INNER), jnp.float32)
        for h in range(HEADS):
            o10 = o10 + t[h * K_Q:(h + 1) * K_Q, :] * (col == h).astype(
                jnp.float32)
        xm = jax.lax.dot_general(o10, wout_ref[...], (((1,), (1,)), ((), ())),
                                 preferred_element_type=jnp.float32)
        xm = xm + outb_ref[...]
        mb = (N_KEEP // 8) * 8
        mr = N_KEEP - mb
        cpin2 = pltpu.make_async_copy(
            out_ref.at[pl.ds(mb, 8), :], bnd_ref, sem)
        cpin2.start()
        cpin2.wait()
        mg_ref[...] = jnp.zeros_like(mg_ref)
        rowm = jax.lax.broadcasted_iota(jnp.int32, (8, 1), 0)
        mg_ref[0:8, :] = jnp.where(rowm < mr, bnd_ref[...],
                                   jnp.zeros((8, DIM), jnp.float32))
        mg_ref[mr:mr + K_Q, :] = xm
        cp2 = pltpu.make_async_copy(
            mg_ref, out_ref.at[pl.ds(mb, 24), :], sem)
        cp2.start()
        cp2.wait()


@jax.jit
def kernel(x, attn, global_q, norm_w, norm_b, to_kv_w, to_q_w, to_out_w,
           to_out_b):
    xs = x[0]
    a2 = attn.reshape(ROWS1, LANES1)
    gq = global_q[0]
    nw = norm_w.reshape(1, DIM)
    nb = norm_b.reshape(1, DIM)
    wk = to_kv_w[:INNER, :]
    wv = to_kv_w[INNER:, :]
    ob = to_out_b.reshape(1, DIM)

    keepf, biasf, dsts, qwn = pl.pallas_call(
        _mask_kernel,
        out_shape=(
            jax.ShapeDtypeStruct((ROWS1, LANES1), jnp.float32),
            jax.ShapeDtypeStruct((ROWS1, LANES1), jnp.float32),
            jax.ShapeDtypeStruct((NBLK, 1), jnp.int32),
            jax.ShapeDtypeStruct((HEADS * K_Q, DIM), jnp.float32),
        ),
    )(a2, gq, nw, nb, to_q_w, wk)

    keep3 = keepf.reshape(NBLK, 1, BLK)
    bias3 = biasf.reshape(NBLK, 1, BLK)
    dflat = dsts.reshape(NBLK)

    OUT_PAD = N_KEEP + BLK + 16
    grid_spec = pltpu.PrefetchScalarGridSpec(
        num_scalar_prefetch=1,
        grid=(NBLK,),
        in_specs=[
            pl.BlockSpec((BLK, DIM), lambda j, d: (j, 0)),
            pl.BlockSpec((1, 1, BLK), lambda j, d: (j, 0, 0)),
            pl.BlockSpec((1, 1, BLK), lambda j, d: (j, 0, 0)),
            pl.BlockSpec((HEADS * K_Q, DIM), lambda j, d: (0, 0)),
            pl.BlockSpec((INNER, DIM), lambda j, d: (0, 0)),
            pl.BlockSpec((DIM, INNER), lambda j, d: (0, 0)),
            pl.BlockSpec((1, DIM), lambda j, d: (0, 0)),
            pl.BlockSpec((1, DIM), lambda j, d: (0, 0)),
            pl.BlockSpec((1, DIM), lambda j, d: (0, 0)),
        ],
        out_specs=pl.BlockSpec(memory_space=pl.ANY),
        scratch_shapes=[
            pltpu.VMEM((HEADS * K_Q, 128), jnp.float32),
            pltpu.VMEM((HEADS * K_Q, 128), jnp.float32),
            pltpu.VMEM((HEADS * K_Q, DIM), jnp.float32),
            pltpu.VMEM((BLK + 8, DIM), jnp.float32),
            pltpu.VMEM((24, DIM), jnp.float32),
            pltpu.VMEM((8, DIM), jnp.float32),
            pltpu.SemaphoreType.DMA,
        ],
    )
    out_buf = pl.pallas_call(
        _main_kernel,
        grid_spec=grid_spec,
        out_shape=jax.ShapeDtypeStruct((OUT_PAD, DIM), jnp.float32),
        compiler_params=pltpu.CompilerParams(
            dimension_semantics=("arbitrary",)),
    )(dflat, xs, keep3, bias3, qwn, wv, to_out_w, ob, nw, nb)

    return out_buf[:N_KEEP + K_Q][None, :, :]
